# Initial kernel scaffold; baseline (speedup 1.0000x reference)
import jax
import jax.numpy as jnp
from jax.experimental import pallas as pl


def kernel(x, enc_w1, enc_b1, enc_w2, enc_b2, codebook, dec_w1, dec_b1, dec_w2, dec_b2):
    raise NotImplementedError("write your pallas kernel here")


# speedup vs baseline: 1.1151x; 1.1151x over previous
import functools

import jax
import jax.numpy as jnp
from jax import lax
from jax.experimental import pallas as pl
from jax.experimental.pallas import tpu as pltpu
from jax.experimental.pallas import tpu_sc as plsc

B, H, W, C = 16, 256, 256, 3
LATENT_DIM = 32
NUM_EMB = 8192
BETA = 0.25

N_TOK = B * 64 * 64
TILE_N = 2048
TILE_K = 1024
N_STEPS = N_TOK // TILE_N
K_STEPS = NUM_EMB // TILE_K

SC_CORES = 2
SC_SUBCORES = 16
SC_WORKERS = SC_CORES * SC_SUBCORES
ROWS_PER_WORKER = N_TOK // SC_WORKERS


def _vq_argmin_body(znorm_ref, flatb_ref, cnorm_ref, cbb_ref, idx_ref,
                    loss_ref, acc_ref):
    step = pl.program_id(0)
    znorm = znorm_ref[...]
    flat_bf = flatb_ref[...]

    big = jnp.float32(3.4e38)

    def chunk(k, carry):
        best_d, best_i = carry
        cnorm = cnorm_ref[k]
        sim = lax.dot_general(
            flat_bf, cbb_ref[k], (((1,), (0,)), ((), ())),
            preferred_element_type=jnp.float32)
        d = znorm + cnorm - 2.0 * sim
        dmin = jnp.min(d, axis=1, keepdims=True)
        lane = lax.broadcasted_iota(jnp.int32, d.shape, 1)
        cand = jnp.where(d == dmin, lane, jnp.int32(NUM_EMB))
        lidx = jnp.min(cand, axis=1, keepdims=True) + k * TILE_K
        take = dmin < best_d
        return jnp.where(take, dmin, best_d), jnp.where(take, lidx, best_i)

    init = (jnp.full((TILE_N, 1), big, jnp.float32),
            jnp.zeros((TILE_N, 1), jnp.int32))
    best_d, best_i = lax.fori_loop(0, K_STEPS, chunk, init)

    idx_ref[...] = best_i
    partial = jnp.sum(best_d).reshape(1, 1)

    @pl.when(step == 0)
    def _():
        acc_ref[...] = jnp.zeros((1, 1), jnp.float32)

    acc_ref[...] += partial

    @pl.when(step == N_STEPS - 1)
    def _():
        loss_ref[...] = acc_ref[...]


def _vq_argmin(znorm, flat_bf, cnorm_chunks, cb_bf_chunks):
    return pl.pallas_call(
        _vq_argmin_body,
        grid=(N_STEPS,),
        in_specs=[
            pl.BlockSpec((TILE_N, 1), lambda i: (i, 0)),
            pl.BlockSpec((TILE_N, LATENT_DIM), lambda i: (i, 0)),
            pl.BlockSpec((K_STEPS, 1, TILE_K), lambda i: (0, 0, 0)),
            pl.BlockSpec((K_STEPS, LATENT_DIM, TILE_K), lambda i: (0, 0, 0)),
        ],
        out_specs=[
            pl.BlockSpec((TILE_N, 1), lambda i: (i, 0)),
            pl.BlockSpec((1, 1), lambda i: (0, 0)),
        ],
        out_shape=[
            jax.ShapeDtypeStruct((N_TOK, 1), jnp.int32),
            jax.ShapeDtypeStruct((1, 1), jnp.float32),
        ],
        scratch_shapes=[pltpu.VMEM((1, 1), jnp.float32)],
    )(znorm, flat_bf, cnorm_chunks, cb_bf_chunks)


def _sc_gather(table, idx):
    mesh = plsc.VectorSubcoreMesh(core_axis_name="c", subcore_axis_name="s")

    @functools.partial(
        pl.kernel, mesh=mesh,
        compiler_params=pltpu.CompilerParams(use_tc_tiling_on_sc=False),
        out_type=jax.ShapeDtypeStruct((N_TOK, LATENT_DIM), jnp.float32),
        scratch_types=[
            pltpu.VMEM((ROWS_PER_WORKER,), jnp.int32),
            pltpu.VMEM((ROWS_PER_WORKER, LATENT_DIM), jnp.float32),
            pltpu.SemaphoreType.DMA,
        ],
    )
    def k(table_hbm, idx_hbm, out_hbm, idx_v, rows_v, sem):
        wid = lax.axis_index("s") * SC_CORES + lax.axis_index("c")
        base = wid * ROWS_PER_WORKER
        pltpu.sync_copy(idx_hbm.at[pl.ds(base, ROWS_PER_WORKER)], idx_v)
        pltpu.async_copy(table_hbm.at[idx_v], rows_v, sem).wait()
        pltpu.sync_copy(rows_v, out_hbm.at[pl.ds(base, ROWS_PER_WORKER)])

    return k(table, idx)


def _conv2d(x, w, b, stride):
    y = lax.conv_general_dilated(
        x, w, window_strides=(stride, stride), padding='SAME',
        dimension_numbers=('NHWC', 'HWIO', 'NHWC'))
    return y + b


def _conv2d_transpose(x, w, b, stride):
    y = lax.conv_transpose(
        x, w, strides=(stride, stride), padding='SAME',
        dimension_numbers=('NHWC', 'HWIO', 'NHWC'))
    return y + b


def kernel(x, enc_w1, enc_b1, enc_w2, enc_b2, codebook,
           dec_w1, dec_b1, dec_w2, dec_b2):
    h = jax.nn.relu(_conv2d(x, enc_w1, enc_b1, stride=2))
    z = _conv2d(h, enc_w2, enc_b2, stride=2)

    flat = z.reshape(N_TOK, LATENT_DIM)
    znorm = jnp.sum(flat ** 2, axis=1, keepdims=True)
    cnorm = jnp.sum(codebook ** 2, axis=0)
    cb_chunks = codebook.reshape(LATENT_DIM, K_STEPS, TILE_K).transpose(1, 0, 2)
    cnorm_chunks = cnorm.reshape(K_STEPS, 1, TILE_K)
    idx2d, loss_sum = _vq_argmin(znorm, flat.astype(jnp.bfloat16),
                                 cnorm_chunks, cb_chunks.astype(jnp.bfloat16))

    quantized = _sc_gather(codebook.T.reshape(NUM_EMB, LATENT_DIM),
                           idx2d.reshape(N_TOK))
    vq_loss = (1.0 + BETA) * loss_sum[0, 0] / (N_TOK * LATENT_DIM)

    q = quantized.reshape(z.shape)
    d = jax.nn.relu(_conv2d_transpose(q, dec_w1, dec_b1, stride=2))
    reconstructions = _conv2d_transpose(d, dec_w2, dec_b2, stride=2)
    return reconstructions, vq_loss
